# Initial kernel scaffold; baseline (speedup 1.0000x reference)
import jax
import jax.numpy as jnp
from jax.experimental import pallas as pl


def kernel(x):
    raise NotImplementedError("write your pallas kernel here")


# speedup vs baseline: 2.4700x; 2.4700x over previous
import functools

import jax
import jax.numpy as jnp
from jax.experimental import pallas as pl
from jax.experimental.pallas import tpu as pltpu

_N = 8192
_R = 64


def _sort_body(x_ref, o_ref):
    z = x_ref[:]
    ii = jax.lax.broadcasted_iota(jnp.int32, (_N, _R), 0)
    k = 2
    while k <= _N:
        j = k // 2
        while j >= 1:
            is_lower = (ii & j) == 0
            up = (ii & k) == 0
            fwd = pltpu.roll(z, _N - j, axis=0)
            bwd = pltpu.roll(z, j, axis=0)
            partner = jnp.where(is_lower, fwd, bwd)
            keep_min = is_lower == up
            z = jnp.where(keep_min, jnp.minimum(z, partner),
                          jnp.maximum(z, partner))
            j //= 2
        k *= 2
    o_ref[:] = z


@functools.partial(jax.jit)
def kernel(x):
    zt = x.T
    out = pl.pallas_call(
        _sort_body,
        out_shape=jax.ShapeDtypeStruct((_N, _R), jnp.float32),
    )(zt)
    return out.T
